# baseline (speedup 1.0000x reference)
import functools
import numpy as np
import jax
import jax.numpy as jnp
from jax import lax
from jax.experimental import pallas as pl
from jax.experimental.pallas import tpu as pltpu
from jax.experimental.pallas import tpu_sc as plsc

N = 10000
D = 128
H = 128
C = 40
CP = 64
E = 320000

NC = 2
NS = 16
LANES = 16
NW = NC * NS
EDGES_PER_TILE = E // NW
CHUNK = 80
NCHUNK = EDGES_PER_TILE // CHUNK
NPAIR = (NCHUNK + 1) // 2
ROWS_PER_TILE = 624

RB = 1000
GRID = N // RB


def _make_sc_spmm(Dd):
    mesh = plsc.VectorSubcoreMesh(
        core_axis_name="c", subcore_axis_name="s",
        num_cores=NC, num_subcores=NS)

    @functools.partial(
        pl.kernel,
        out_type=jax.ShapeDtypeStruct((NC, N, Dd), jnp.float32),
        mesh=mesh,
        compiler_params=pltpu.CompilerParams(use_tc_tiling_on_sc=False),
        scratch_types=[
            pltpu.VMEM_SHARED((N, Dd), jnp.float32),
            pltpu.VMEM((EDGES_PER_TILE,), jnp.int32),
            pltpu.VMEM((2, CHUNK), jnp.int32),
            pltpu.VMEM((EDGES_PER_TILE,), jnp.float32),
            pltpu.VMEM((2, CHUNK, Dd), jnp.bfloat16),
            pltpu.VMEM((2, CHUNK, Dd), jnp.float32),
            pltpu.SemaphoreType.DMA,
            pltpu.SemaphoreType.DMA,
            pltpu.SemaphoreType.DMA,
            pltpu.SemaphoreType.DMA,
            pltpu.SemaphoreType.DMA,
            pltpu.SemaphoreType.DMA,
        ],
    )
    def spmm(table, src, dst, w, zeros, out, acc, src_v, dst_v, w_v,
             rows_g, rows_f, sem_g0, sem_g1, sem_d0, sem_d1, sem_s0, sem_s1):
        sem_g = [sem_g0, sem_g1]
        sem_d = [sem_d0, sem_d1]
        sem_s = [sem_s0, sem_s1]
        c = lax.axis_index("c")
        s = lax.axis_index("s")
        tid = c * NS + s
        r0 = s * ROWS_PER_TILE
        tail0 = NS * ROWS_PER_TILE
        tail = N - tail0
        pltpu.sync_copy(zeros.at[pl.ds(r0, ROWS_PER_TILE)],
                        acc.at[pl.ds(r0, ROWS_PER_TILE)])

        @pl.when(s == NS - 1)
        def _():
            pltpu.sync_copy(zeros.at[pl.ds(tail0, tail)],
                            acc.at[pl.ds(tail0, tail)])
        plsc.subcore_barrier()

        base0 = tid * EDGES_PER_TILE
        pltpu.sync_copy(src.at[pl.ds(base0, EDGES_PER_TILE)], src_v)
        pltpu.sync_copy(w.at[pl.ds(base0, EDGES_PER_TILE)], w_v)

        def start_gather(k, p):
            pltpu.async_copy(
                table.at[src_v.at[pl.ds(k * CHUNK, CHUNK)]],
                rows_g.at[p], sem_g[p])
            pltpu.async_copy(
                dst.at[pl.ds(base0 + k * CHUNK, CHUNK)],
                dst_v.at[p], sem_d[p])

        def wait_gather(p):
            pltpu.make_async_copy(
                table.at[pl.ds(0, CHUNK)], rows_g.at[p], sem_g[p]).wait()

        def wait_dst(p):
            pltpu.make_async_copy(
                dst.at[pl.ds(0, CHUNK)], dst_v.at[p], sem_d[p]).wait()

        def drain_scatter(p):
            pltpu.make_async_copy(
                out.at[0, pl.ds(0, CHUNK)], rows_f.at[p], sem_s[p]).wait()

        def do_step(k, p):
            q = 1 - p

            @pl.when(k + 1 < NCHUNK)
            def _():
                start_gather(k + 1, q)
            wait_gather(p)
            wait_dst(p)

            @pl.when(k >= 2)
            def _():
                drain_scatter(p)
            rg = rows_g.at[p]
            rf = rows_f.at[p]

            def group_body(g, carry2):
                w16 = w_v[pl.ds(k * CHUNK + g * LANES, LANES)]
                for i in range(LANES):
                    e = g * LANES + i
                    wb = lax.gather(
                        w16, jnp.full((LANES, 1), i, jnp.int32),
                        lax.GatherDimensionNumbers(
                            offset_dims=(), collapsed_slice_dims=(0,),
                            start_index_map=(0,)),
                        slice_sizes=(1,),
                        mode=lax.GatherScatterMode.PROMISE_IN_BOUNDS)
                    for j in range(Dd // 32):
                        fall = rg[e, pl.ds(j * 32, 32)].astype(jnp.float32)
                        rf[e, pl.ds(j * 32, LANES)] = fall[:LANES] * wb
                        rf[e, pl.ds(j * 32 + LANES, LANES)] = \
                            fall[LANES:] * wb
                return carry2

            lax.fori_loop(0, CHUNK // LANES, group_body, 0)
            pltpu.async_copy(rf, acc.at[dst_v.at[p]], sem_s[p], add=True)

        start_gather(0, 0)

        def pair_body(kp, carry):
            a = 2 * kp
            do_step(a, 0)

            @pl.when(a + 1 < NCHUNK)
            def _():
                do_step(a + 1, 1)
            return carry

        lax.fori_loop(0, NPAIR, pair_body, 0)
        drain_scatter(1 - (NCHUNK - 1) % 2)
        drain_scatter((NCHUNK - 1) % 2)
        plsc.subcore_barrier()
        pltpu.sync_copy(acc.at[pl.ds(r0, ROWS_PER_TILE)],
                        out.at[c, pl.ds(r0, ROWS_PER_TILE)])

        @pl.when(s == NS - 1)
        def _():
            pltpu.sync_copy(acc.at[pl.ds(tail0, tail)],
                            out.at[c, pl.ds(tail0, tail)])

    return spmm


_sc_spmm_h = _make_sc_spmm(H)
_sc_spmm_c = _make_sc_spmm(CP)


def _mm_body(x_ref, w_ref, o_ref):
    z = jnp.dot(x_ref[...], w_ref[...], preferred_element_type=jnp.float32)
    o_ref[...] = z.astype(jnp.bfloat16)


def _layer1_mm(x, W1):
    return pl.pallas_call(
        _mm_body,
        grid=(GRID,),
        in_specs=[pl.BlockSpec((RB, D), lambda i: (i, 0)),
                  pl.BlockSpec((D, H), lambda i: (0, 0))],
        out_specs=pl.BlockSpec((RB, H), lambda i: (i, 0)),
        out_shape=jax.ShapeDtypeStruct((N, H), jnp.bfloat16),
    )(x, W1)


def _layer2_body(p0_ref, p1_ref, b1_ref, w2_ref, o_ref):
    h = jax.nn.relu(p0_ref[...] + p1_ref[...] + b1_ref[...])
    z = jnp.dot(h, w2_ref[...], preferred_element_type=jnp.float32)
    o_ref[...] = z.astype(jnp.bfloat16)


def _layer2_mm(p0, p1, b1p, w2f):
    return pl.pallas_call(
        _layer2_body,
        grid=(GRID,),
        in_specs=[pl.BlockSpec((RB, H), lambda i: (i, 0)),
                  pl.BlockSpec((RB, H), lambda i: (i, 0)),
                  pl.BlockSpec((1, H), lambda i: (0, 0)),
                  pl.BlockSpec((H, CP), lambda i: (0, 0))],
        out_specs=pl.BlockSpec((RB, CP), lambda i: (i, 0)),
        out_shape=jax.ShapeDtypeStruct((N, CP), jnp.bfloat16),
    )(p0, p1, b1p, w2f)


def _logsoftmax_body(q0_ref, q1_ref, b2_ref, o_ref):
    z = q0_ref[...] + q1_ref[...] + b2_ref[...]
    col = lax.broadcasted_iota(jnp.int32, (RB, CP), 1)
    z = jnp.where(col < C, z, -1e30)
    m = jnp.max(z, axis=-1, keepdims=True)
    lse = jnp.log(jnp.sum(jnp.exp(z - m), axis=-1, keepdims=True)) + m
    o_ref[...] = (z - lse)[:, :C]


def _final(q0, q1, b2p):
    return pl.pallas_call(
        _logsoftmax_body,
        grid=(GRID,),
        in_specs=[pl.BlockSpec((RB, CP), lambda i: (i, 0)),
                  pl.BlockSpec((RB, CP), lambda i: (i, 0)),
                  pl.BlockSpec((1, CP), lambda i: (0, 0))],
        out_specs=pl.BlockSpec((RB, C), lambda i: (i, 0)),
        out_shape=jax.ShapeDtypeStruct((N, C), jnp.float32),
    )(q0, q1, b2p)


def kernel(x, edge_index, edge_weight, W1, b1, W2, b2):
    src = edge_index[1]
    dst = edge_index[0]

    support = _layer1_mm(x, W1)
    zeros_h = jnp.zeros((N, H), jnp.float32)
    parts1 = _sc_spmm_h(support, src, dst, edge_weight, zeros_h)

    w2p = jnp.zeros((H, CP), jnp.float32).at[:, :C].set(W2)
    sup2 = _layer2_mm(parts1[0], parts1[1], b1.reshape(1, H), w2p)

    zeros_c = jnp.zeros((N, CP), jnp.float32)
    parts2 = _sc_spmm_c(sup2, src, dst, edge_weight, zeros_c)

    b2p = jnp.zeros((1, CP), jnp.float32).at[0, :C].set(b2)
    return _final(parts2[0], parts2[1], b2p)


# speedup vs baseline: 1.7202x; 1.7202x over previous
import functools
import jax
import jax.numpy as jnp
from jax import lax
from jax.experimental import pallas as pl
from jax.experimental.pallas import tpu as pltpu
from jax.experimental.pallas import tpu_sc as plsc

N = 10000
D = 128
H = 128
C = 40
CP = 48
E = 320000

NC = 2
NS = 16
LANES = 16
NW = NC * NS
EDGES_PER_TILE = E // NW
CHUNK = 80
NCHUNK = EDGES_PER_TILE // CHUNK
NPAIR = (NCHUNK + 1) // 2
ROWS_PER_TILE = 624

RB = 1000
GRID = N // RB


def _make_sc_spmm(Dd):
    mesh = plsc.VectorSubcoreMesh(
        core_axis_name="c", subcore_axis_name="s",
        num_cores=NC, num_subcores=NS)

    @functools.partial(
        pl.kernel,
        out_type=(jax.ShapeDtypeStruct((N, Dd), jnp.float32),
                  jax.ShapeDtypeStruct((N, Dd), jnp.float32)),
        mesh=mesh,
        compiler_params=pltpu.CompilerParams(use_tc_tiling_on_sc=False),
        scratch_types=[
            pltpu.VMEM_SHARED((N, Dd), jnp.float32),
            pltpu.VMEM((EDGES_PER_TILE,), jnp.int32),
            pltpu.VMEM((2, CHUNK), jnp.int32),
            pltpu.VMEM((EDGES_PER_TILE,), jnp.float32),
            pltpu.VMEM((2, CHUNK, Dd), jnp.float32),
            pltpu.SemaphoreType.DMA,
            pltpu.SemaphoreType.DMA,
            pltpu.SemaphoreType.DMA,
            pltpu.SemaphoreType.DMA,
            pltpu.SemaphoreType.DMA,
            pltpu.SemaphoreType.DMA,
        ],
    )
    def spmm(table, ei, w, zeros, out0, out1, acc, src_v, dst_v, w_v,
             rows_v, sem_g0, sem_g1, sem_d0, sem_d1, sem_s0, sem_s1):
        sem_g = [sem_g0, sem_g1]
        sem_d = [sem_d0, sem_d1]
        sem_s = [sem_s0, sem_s1]
        c = lax.axis_index("c")
        s = lax.axis_index("s")
        tid = c * NS + s
        r0 = s * ROWS_PER_TILE
        tail0 = NS * ROWS_PER_TILE
        tail = N - tail0
        pltpu.sync_copy(zeros.at[pl.ds(r0, ROWS_PER_TILE)],
                        acc.at[pl.ds(r0, ROWS_PER_TILE)])

        @pl.when(s == NS - 1)
        def _():
            pltpu.sync_copy(zeros.at[pl.ds(tail0, tail)],
                            acc.at[pl.ds(tail0, tail)])
        plsc.subcore_barrier()

        base0 = tid * EDGES_PER_TILE
        pltpu.sync_copy(ei.at[1, pl.ds(base0, EDGES_PER_TILE)], src_v)
        pltpu.sync_copy(w.at[pl.ds(base0, EDGES_PER_TILE)], w_v)

        def start_gather(k, p):
            pltpu.async_copy(
                table.at[src_v.at[pl.ds(k * CHUNK, CHUNK)]],
                rows_v.at[p], sem_g[p])
            pltpu.async_copy(
                ei.at[0, pl.ds(base0 + k * CHUNK, CHUNK)],
                dst_v.at[p], sem_d[p])

        def wait_gather(p):
            pltpu.make_async_copy(
                table.at[pl.ds(0, CHUNK)], rows_v.at[p], sem_g[p]).wait()

        def wait_dst(p):
            pltpu.make_async_copy(
                ei.at[0, pl.ds(0, CHUNK)], dst_v.at[p], sem_d[p]).wait()

        def drain_scatter(p):
            pltpu.make_async_copy(
                out0.at[pl.ds(0, CHUNK)], rows_v.at[p], sem_s[p]).wait()

        def do_step(k, p):
            q = 1 - p

            @pl.when(k + 1 < NCHUNK)
            def _():
                @pl.when(k >= 1)
                def _():
                    drain_scatter(q)
                start_gather(k + 1, q)
            wait_gather(p)
            wait_dst(p)
            rp = rows_v.at[p]

            def group_body(g, carry2):
                w16 = w_v[pl.ds(k * CHUNK + g * LANES, LANES)]
                for i in range(LANES):
                    e = g * LANES + i
                    wb = lax.gather(
                        w16, jnp.full((LANES, 1), i, jnp.int32),
                        lax.GatherDimensionNumbers(
                            offset_dims=(), collapsed_slice_dims=(0,),
                            start_index_map=(0,)),
                        slice_sizes=(1,),
                        mode=lax.GatherScatterMode.PROMISE_IN_BOUNDS)
                    for j in range(Dd // LANES):
                        sl = pl.ds(j * LANES, LANES)
                        rp[e, sl] = rp[e, sl] * wb
                return carry2

            lax.fori_loop(0, CHUNK // LANES, group_body, 0)
            pltpu.async_copy(rp, acc.at[dst_v.at[p]], sem_s[p], add=True)

        start_gather(0, 0)

        def pair_body(kp, carry):
            a = 2 * kp
            do_step(a, 0)

            @pl.when(a + 1 < NCHUNK)
            def _():
                do_step(a + 1, 1)
            return carry

        lax.fori_loop(0, NPAIR, pair_body, 0)
        drain_scatter(1 - (NCHUNK - 1) % 2)
        drain_scatter((NCHUNK - 1) % 2)
        plsc.subcore_barrier()

        @pl.when(c == 0)
        def _():
            pltpu.sync_copy(acc.at[pl.ds(r0, ROWS_PER_TILE)],
                            out0.at[pl.ds(r0, ROWS_PER_TILE)])

            @pl.when(s == NS - 1)
            def _():
                pltpu.sync_copy(acc.at[pl.ds(tail0, tail)],
                                out0.at[pl.ds(tail0, tail)])

        @pl.when(c == 1)
        def _():
            pltpu.sync_copy(acc.at[pl.ds(r0, ROWS_PER_TILE)],
                            out1.at[pl.ds(r0, ROWS_PER_TILE)])

            @pl.when(s == NS - 1)
            def _():
                pltpu.sync_copy(acc.at[pl.ds(tail0, tail)],
                                out1.at[pl.ds(tail0, tail)])

    return spmm


_sc_spmm_h = _make_sc_spmm(H)
_sc_spmm_c = _make_sc_spmm(CP)


def _mm_body(x_ref, w_ref, o_ref):
    o_ref[...] = jnp.dot(x_ref[...].astype(jnp.bfloat16),
                         w_ref[...].astype(jnp.bfloat16),
                         preferred_element_type=jnp.float32)


def _layer1_mm(x, W1):
    return pl.pallas_call(
        _mm_body,
        grid=(GRID,),
        in_specs=[pl.BlockSpec((RB, D), lambda i: (i, 0)),
                  pl.BlockSpec((D, H), lambda i: (0, 0))],
        out_specs=pl.BlockSpec((RB, H), lambda i: (i, 0)),
        out_shape=jax.ShapeDtypeStruct((N, H), jnp.float32),
    )(x, W1)


def _layer2_body(p0_ref, p1_ref, b1_ref, w2_ref, o_ref):
    h = jax.nn.relu(p0_ref[...] + p1_ref[...] + b1_ref[...])
    o_ref[...] = jnp.dot(h.astype(jnp.bfloat16),
                         w2_ref[...].astype(jnp.bfloat16),
                         preferred_element_type=jnp.float32)


def _layer2_mm(p0, p1, b1p, w2p):
    return pl.pallas_call(
        _layer2_body,
        grid=(GRID,),
        in_specs=[pl.BlockSpec((RB, H), lambda i: (i, 0)),
                  pl.BlockSpec((RB, H), lambda i: (i, 0)),
                  pl.BlockSpec((1, H), lambda i: (0, 0)),
                  pl.BlockSpec((H, CP), lambda i: (0, 0))],
        out_specs=pl.BlockSpec((RB, CP), lambda i: (i, 0)),
        out_shape=jax.ShapeDtypeStruct((N, CP), jnp.float32),
    )(p0, p1, b1p, w2p)


def _logsoftmax_body(q0_ref, q1_ref, b2_ref, o_ref):
    z = q0_ref[...] + q1_ref[...] + b2_ref[...]
    col = lax.broadcasted_iota(jnp.int32, (RB, CP), 1)
    z = jnp.where(col < C, z, -1e30)
    m = jnp.max(z, axis=-1, keepdims=True)
    lse = jnp.log(jnp.sum(jnp.exp(z - m), axis=-1, keepdims=True)) + m
    o_ref[...] = (z - lse)[:, :C]


def _final(q0, q1, b2p):
    return pl.pallas_call(
        _logsoftmax_body,
        grid=(GRID,),
        in_specs=[pl.BlockSpec((RB, CP), lambda i: (i, 0)),
                  pl.BlockSpec((RB, CP), lambda i: (i, 0)),
                  pl.BlockSpec((1, CP), lambda i: (0, 0))],
        out_specs=pl.BlockSpec((RB, C), lambda i: (i, 0)),
        out_shape=jax.ShapeDtypeStruct((N, C), jnp.float32),
    )(q0, q1, b2p)


def kernel(x, edge_index, edge_weight, W1, b1, W2, b2):
    support = _layer1_mm(x, W1)
    zeros_h = jnp.zeros((N, H), jnp.float32)
    p0, p1 = _sc_spmm_h(support, edge_index, edge_weight, zeros_h)

    w2p = jnp.zeros((H, CP), jnp.float32).at[:, :C].set(W2)
    sup2 = _layer2_mm(p0, p1, b1.reshape(1, H), w2p)

    zeros_c = jnp.zeros((N, CP), jnp.float32)
    q0, q1 = _sc_spmm_c(sup2, edge_index, edge_weight, zeros_c)

    b2p = jnp.zeros((1, CP), jnp.float32).at[0, :C].set(b2)
    return _final(q0, q1, b2p)
